# baseline (speedup 1.0000x reference)
import functools

import jax
import jax.numpy as jnp
from jax.experimental import pallas as pl
from jax.experimental.pallas import tpu as pltpu

_N = 256
_S = 48
_OFF = 24
_W = 22
_G = (_N + _W - 1) // _W
_CPAD = _G * _W


def _build_u_kernel(th_ref, ph_ref, ur_ref, ui_ref,
                    dpr, dpi, vtr, vti,
                    c_ar, c_ai, c_br, c_bi):
    n = _N
    m = n // 2
    f32 = jnp.float32

    ct = jnp.cos(th_ref[...])
    st = jnp.sin(th_ref[...])
    cp = jnp.cos(ph_ref[...])
    sp = jnp.sin(ph_ref[...])
    ar = 0.5 * ((ct - 1.0) * cp - st * sp)
    ai = 0.5 * ((ct - 1.0) * sp + st * cp)
    br = -0.5 * st
    bi = 0.5 * (ct + 1.0)
    cr = br * cp - bi * sp
    ci = br * sp + bi * cp
    dr = 0.5 * (1.0 - ct)
    di = -0.5 * st

    erow = jax.lax.broadcasted_iota(jnp.int32, (m, n), 0)
    elane = jax.lax.broadcasted_iota(jnp.int32, (m, n), 1)
    emat = ((elane == 2 * erow) | (elane == 2 * erow + 1)).astype(f32)
    crow = jax.lax.broadcasted_iota(jnp.int32, (n, n), 0)
    lane = jax.lax.broadcasted_iota(jnp.int32, (n, n), 1)
    col_even = crow % 2 == 0
    lane_even = lane % 2 == 0
    untouched = (~col_even) & ((lane == 0) | (lane == n - 1))
    zero = jnp.zeros((n, n), f32)
    one = jnp.ones((n, n), f32)

    def expand(top, bot, ident):
        rt = jnp.dot(top, emat, preferred_element_type=f32,
                     precision=jax.lax.Precision.HIGHEST)
        rb = jnp.dot(bot, emat, preferred_element_type=f32,
                     precision=jax.lax.Precision.HIGHEST)
        even_plane = jnp.where(lane_even, rt, rb)
        odd_plane = jnp.where(lane_even, pltpu.roll(rb, 2, 1), rt)
        out = jnp.where(col_even, even_plane, odd_plane)
        return jnp.where(untouched, ident, out)

    c_ar[0:n] = expand(ar, dr, one).reshape(n, 1, n)
    c_ai[0:n] = expand(ai, di, zero).reshape(n, 1, n)
    c_br[0:n] = expand(br, cr, zero).reshape(n, 1, n)
    c_bi[0:n] = expand(bi, ci, zero).reshape(n, 1, n)
    pad = _CPAD - n
    c_ar[n:_CPAD] = jnp.ones((pad, 1, n), f32)
    c_ai[n:_CPAD] = jnp.zeros((pad, 1, n), f32)
    c_br[n:_CPAD] = jnp.zeros((pad, 1, n), f32)
    c_bi[n:_CPAD] = jnp.zeros((pad, 1, n), f32)

    dpr[...] = jnp.zeros((n, n), f32)
    dpi[...] = jnp.zeros((n, n), f32)
    dd = jax.lax.broadcasted_iota(jnp.int32, (_S, n), 0)
    ident_band = (dd == _OFF).astype(f32)
    top_if_even = (jax.lax.broadcasted_iota(jnp.int32, (1, n), 1) % 2) == 0

    def column_step(c, topm, d_r, d_i):
        a_r = c_ar[pl.ds(c, 1)].reshape(1, n)
        a_i = c_ai[pl.ds(c, 1)].reshape(1, n)
        b_r = c_br[pl.ds(c, 1)].reshape(1, n)
        b_i = c_bi[pl.ds(c, 1)].reshape(1, n)
        wt_r = pltpu.roll(pltpu.roll(d_r, 1, 0), n - 1, 1)
        wt_i = pltpu.roll(pltpu.roll(d_i, 1, 0), n - 1, 1)
        wb_r = pltpu.roll(pltpu.roll(d_r, _S - 1, 0), 1, 1)
        wb_i = pltpu.roll(pltpu.roll(d_i, _S - 1, 0), 1, 1)
        w_r = jnp.where(topm, wt_r, wb_r)
        w_i = jnp.where(topm, wt_i, wb_i)
        n_r = a_r * d_r - a_i * d_i + b_r * w_r - b_i * w_i
        n_i = a_r * d_i + a_i * d_r + b_r * w_i + b_i * w_r
        return n_r, n_i

    def group_body(g, carry):
        dpr[0:_S, :] = ident_band
        dpi[0:_S, :] = jnp.zeros((_S, n), f32)

        def body(t, carry2):
            d_r = dpr[0:_S, :]
            d_i = dpi[0:_S, :]
            c = g * _W + 2 * t
            d_r, d_i = column_step(c, top_if_even, d_r, d_i)
            d_r, d_i = column_step(c + 1, ~top_if_even, d_r, d_i)
            dpr[0:_S, :] = d_r
            dpi[0:_S, :] = d_i
            return carry2

        jax.lax.fori_loop(0, _W // 2, body, 0)

        g_r = pltpu.roll(dpr[...].T, (n - _OFF) % n, 1, stride=1, stride_axis=0)
        g_i = pltpu.roll(dpi[...].T, (n - _OFF) % n, 1, stride=1, stride_axis=0)

        @pl.when(g == 0)
        def _():
            vtr[...] = g_r
            vti[...] = g_i

        @pl.when(g > 0)
        def _():
            v_r = vtr[...]
            v_i = vti[...]
            hp = jax.lax.Precision.HIGHEST
            nv_r = (jnp.dot(g_r, v_r, preferred_element_type=f32, precision=hp)
                    - jnp.dot(g_i, v_i, preferred_element_type=f32, precision=hp))
            nv_i = (jnp.dot(g_r, v_i, preferred_element_type=f32, precision=hp)
                    + jnp.dot(g_i, v_r, preferred_element_type=f32, precision=hp))
            vtr[...] = nv_r
            vti[...] = nv_i

        return carry

    jax.lax.fori_loop(0, _G, group_body, 0)

    ur_ref[...] = vtr[...]
    ui_ref[...] = vti[...]


def _apply_kernel(x_ref, ur_ref, ui_ref, out_ref):
    hp = jax.lax.Precision.HIGHEST
    yr = jnp.dot(ur_ref[...], x_ref[...], preferred_element_type=jnp.float32,
                 precision=hp)
    yi = jnp.dot(ui_ref[...], x_ref[...], preferred_element_type=jnp.float32,
                 precision=hp)
    out_ref[...] = yr * yr + yi * yi


@functools.partial(jax.jit, static_argnames=("interpret",))
def kernel(x, thetas, phis, interpret=False):
    n, b = x.shape
    f32 = jnp.float32

    ur, ui = pl.pallas_call(
        _build_u_kernel,
        out_shape=[jax.ShapeDtypeStruct((n, n), f32)] * 2,
        scratch_shapes=[pltpu.VMEM((n, n), f32)] * 4
        + [pltpu.VMEM((_CPAD, 1, n), f32)] * 4,
        interpret=interpret,
    )(thetas, phis)

    bblk = 512
    out = pl.pallas_call(
        _apply_kernel,
        grid=(b // bblk,),
        in_specs=[
            pl.BlockSpec((n, bblk), lambda i: (0, i)),
            pl.BlockSpec((n, n), lambda i: (0, 0)),
            pl.BlockSpec((n, n), lambda i: (0, 0)),
        ],
        out_specs=pl.BlockSpec((n, bblk), lambda i: (0, i)),
        out_shape=jax.ShapeDtypeStruct((n, b), f32),
        compiler_params=pltpu.CompilerParams(
            dimension_semantics=("arbitrary",),
        ),
        interpret=interpret,
    )(x, ur, ui)
    return out


# speedup vs baseline: 1.1976x; 1.1976x over previous
import functools

import jax
import jax.numpy as jnp
from jax.experimental import pallas as pl
from jax.experimental.pallas import tpu as pltpu


def _build_u(th_ref, ph_ref, urs, uis,
             uer, uei, uor, uoi,
             car, cai, cbr, cbi, ccr, cci, cdr, cdi):
    n, m = uer.shape
    f32 = jnp.float32

    jj = jax.lax.broadcasted_iota(jnp.int32, (n, m), 0)
    kk = jax.lax.broadcasted_iota(jnp.int32, (n, m), 1)
    uer[...] = (jj == 2 * kk).astype(f32)
    uei[...] = jnp.zeros((n, m), f32)
    uor[...] = (jj == 2 * kk + 1).astype(f32)
    uoi[...] = jnp.zeros((n, m), f32)

    ct = jnp.cos(th_ref[...])
    st = jnp.sin(th_ref[...])
    cp = jnp.cos(ph_ref[...])
    sp = jnp.sin(ph_ref[...])
    ar = 0.5 * ((ct - 1.0) * cp - st * sp)
    ai = 0.5 * ((ct - 1.0) * sp + st * cp)
    br = -0.5 * st
    bi = 0.5 * (ct + 1.0)
    cr = br * cp - bi * sp
    ci = br * sp + bi * cp
    dr = 0.5 * (1.0 - ct)
    di = -0.5 * st
    crow = jax.lax.broadcasted_iota(jnp.int32, (n, m), 0)
    lane = jax.lax.broadcasted_iota(jnp.int32, (n, m), 1)
    edge = (crow % 2 == 1) & (lane == m - 1)
    zero = jnp.zeros((n, m), f32)
    one = jnp.ones((n, m), f32)
    car[...] = jnp.where(edge, one, ar).reshape(n, 1, m)
    cai[...] = jnp.where(edge, zero, ai).reshape(n, 1, m)
    cbr[...] = jnp.where(edge, zero, br).reshape(n, 1, m)
    cbi[...] = jnp.where(edge, zero, bi).reshape(n, 1, m)
    ccr[...] = jnp.where(edge, zero, cr).reshape(n, 1, m)
    cci[...] = jnp.where(edge, zero, ci).reshape(n, 1, m)
    cdr[...] = jnp.where(edge, one, dr).reshape(n, 1, m)
    cdi[...] = jnp.where(edge, zero, di).reshape(n, 1, m)

    def mix(c, t_r, t_i, w_r, w_i):
        a_r = car[pl.ds(c, 1)].reshape(1, m)
        a_i = cai[pl.ds(c, 1)].reshape(1, m)
        b_r = cbr[pl.ds(c, 1)].reshape(1, m)
        b_i = cbi[pl.ds(c, 1)].reshape(1, m)
        c_r = ccr[pl.ds(c, 1)].reshape(1, m)
        c_i = cci[pl.ds(c, 1)].reshape(1, m)
        d_r = cdr[pl.ds(c, 1)].reshape(1, m)
        d_i = cdi[pl.ds(c, 1)].reshape(1, m)
        nt_r = a_r * t_r - a_i * t_i + b_r * w_r - b_i * w_i
        nt_i = a_r * t_i + a_i * t_r + b_r * w_i + b_i * w_r
        nb_r = c_r * t_r - c_i * t_i + d_r * w_r - d_i * w_i
        nb_i = c_r * t_i + c_i * t_r + d_r * w_i + d_i * w_r
        return nt_r, nt_i, nb_r, nb_i

    def body(k, carry):
        nt_r, nt_i, nb_r, nb_i = mix(2 * k, uer[...], uei[...], uor[...], uoi[...])
        uer[...] = nt_r
        uei[...] = nt_i
        w_r = pltpu.roll(nt_r, m - 1, 1)
        w_i = pltpu.roll(nt_i, m - 1, 1)
        ot_r, ot_i, ob_r, ob_i = mix(2 * k + 1, nb_r, nb_i, w_r, w_i)
        uor[...] = ot_r
        uoi[...] = ot_i
        uer[...] = pltpu.roll(ob_r, 1, 1)
        uei[...] = pltpu.roll(ob_i, 1, 1)
        return carry

    jax.lax.fori_loop(0, m, body, 0)

    rr = jax.lax.broadcasted_iota(jnp.int32, (n, n), 0)
    qq = jax.lax.broadcasted_iota(jnp.int32, (n, n), 1)
    perm = (qq == (rr // 2 + m * (rr % 2))).astype(f32)
    s_r = jnp.concatenate([uer[...], uor[...]], axis=1)
    s_i = jnp.concatenate([uei[...], uoi[...]], axis=1)
    dn = (((1,), (1,)), ((), ()))
    urs[...] = jax.lax.dot_general(perm, s_r, dn, preferred_element_type=f32)
    uis[...] = jax.lax.dot_general(perm, s_i, dn, preferred_element_type=f32)


def _mesh_kernel(th_ref, ph_ref, x_ref, out_ref, urs, uis, *scratch):
    @pl.when(pl.program_id(0) == 0)
    def _():
        _build_u(th_ref, ph_ref, urs, uis, *scratch)

    yr = jnp.dot(urs[...], x_ref[...], preferred_element_type=jnp.float32)
    yi = jnp.dot(uis[...], x_ref[...], preferred_element_type=jnp.float32)
    out_ref[...] = yr * yr + yi * yi


@functools.partial(jax.jit, static_argnames=("interpret",))
def kernel(x, thetas, phis, interpret=False):
    n, b = x.shape
    m = n // 2
    f32 = jnp.float32

    bblk = 512
    out = pl.pallas_call(
        _mesh_kernel,
        grid=(b // bblk,),
        in_specs=[
            pl.BlockSpec((n, m), lambda i: (0, 0)),
            pl.BlockSpec((n, m), lambda i: (0, 0)),
            pl.BlockSpec((n, bblk), lambda i: (0, i)),
        ],
        out_specs=pl.BlockSpec((n, bblk), lambda i: (0, i)),
        out_shape=jax.ShapeDtypeStruct((n, b), f32),
        scratch_shapes=[pltpu.VMEM((n, n), f32)] * 2
        + [pltpu.VMEM((n, m), f32)] * 4
        + [pltpu.VMEM((n, 1, m), f32)] * 8,
        compiler_params=pltpu.CompilerParams(
            dimension_semantics=("arbitrary",),
        ),
        interpret=interpret,
    )(thetas, phis, x)
    return out
